# baseline (speedup 1.0000x reference)
import functools

import jax
import jax.numpy as jnp
from jax import lax
from jax.experimental import pallas as pl
from jax.experimental.pallas import tpu as pltpu
from jax.experimental.pallas import tpu_sc as plsc

N_NODES = 10000
N_EDGES = 320000
IN_FEATS = 128
HIDDEN = 128
NUM_CLASSES = 40
CLS_PAD = 48

NC = 2
NS = 16
NW = NC * NS
EDGES_PER_TILE = N_EDGES // NW
CHUNK = 80
NCHUNKS = EDGES_PER_TILE // CHUNK
N_PAD = 10240
CHUNK_AGG = 72
NCHUNKS_AGG = 141
ROWS_PER_TILE = N_PAD // NS

_mesh = plsc.VectorSubcoreMesh(core_axis_name="c", subcore_axis_name="s")
_sc_params = pltpu.CompilerParams(use_tc_tiling_on_sc=False)


@functools.partial(
    pl.kernel,
    out_type=[
        jax.ShapeDtypeStruct((NC, N_PAD, 16), jnp.float32),
        jax.ShapeDtypeStruct((NC, N_PAD, 16), jnp.float32),
    ],
    mesh=_mesh,
    scratch_types=[
        pltpu.VMEM((NCHUNKS, CHUNK), jnp.int32),
        pltpu.VMEM((NCHUNKS, CHUNK), jnp.int32),
        pltpu.VMEM((CHUNK, 16), jnp.float32),
        pltpu.VMEM_SHARED((N_PAD, 16), jnp.float32),
        pltpu.SemaphoreType.DMA,
    ],
    compiler_params=_sc_params,
)
def _sc_degrees(src_hbm, dst_hbm, ones_hbm, zeros_hbm, od_out, id_out,
                sidx, didx, ones_v, deg_sh, sem):
    c = lax.axis_index("c")
    s = lax.axis_index("s")
    wid = s * NC + c

    pltpu.sync_copy(src_hbm.at[wid], sidx)
    pltpu.sync_copy(dst_hbm.at[wid], didx)
    pltpu.sync_copy(ones_hbm, ones_v)
    row0 = s * ROWS_PER_TILE

    for idx_ref, out_ref in ((sidx, od_out), (didx, id_out)):
        pltpu.sync_copy(zeros_hbm, deg_sh.at[pl.ds(row0, ROWS_PER_TILE)])
        plsc.subcore_barrier()

        @pl.loop(0, NCHUNKS)
        def _(j, idx_ref=idx_ref):
            pltpu.async_copy(ones_v, deg_sh.at[idx_ref.at[j]], sem, add=True)

            @pl.when(j >= 4)
            def _():
                pltpu.make_async_copy(ones_v, deg_sh.at[idx_ref.at[j - 4]],
                                      sem).wait()

        @pl.loop(NCHUNKS - 4, NCHUNKS)
        def _(j, idx_ref=idx_ref):
            pltpu.make_async_copy(ones_v, deg_sh.at[idx_ref.at[j]], sem).wait()

        plsc.subcore_barrier()
        pltpu.sync_copy(deg_sh.at[pl.ds(row0, ROWS_PER_TILE)],
                        out_ref.at[c, pl.ds(row0, ROWS_PER_TILE)])
        plsc.subcore_barrier()


_NBUF = 3


def _make_sc_aggregate(width):
    @functools.partial(
        pl.kernel,
        out_type=jax.ShapeDtypeStruct((NC, N_PAD, width), jnp.float32),
        mesh=_mesh,
        scratch_types=[
            pltpu.VMEM((NCHUNKS_AGG, CHUNK_AGG), jnp.int32),
            pltpu.VMEM((NCHUNKS_AGG, CHUNK_AGG), jnp.int32),
            pltpu.VMEM((CHUNK_AGG, width), jnp.float32),
            pltpu.VMEM((CHUNK_AGG, width), jnp.float32),
            pltpu.VMEM((CHUNK_AGG, width), jnp.float32),
            pltpu.SemaphoreType.DMA,
            pltpu.SemaphoreType.DMA,
            pltpu.SemaphoreType.DMA,
            pltpu.SemaphoreType.DMA,
            pltpu.SemaphoreType.DMA,
            pltpu.SemaphoreType.DMA,
            pltpu.VMEM_SHARED((N_PAD, width), jnp.float32),
        ],
        compiler_params=_sc_params,
    )
    def _sc_aggregate(h_hbm, src_hbm, dst_hbm, zeros_hbm, out_hbm,
                      sidx, didx, b0, b1, b2,
                      g0, g1, g2, s0, s1, s2, agg_sh):
        bufs = [b0, b1, b2]
        gsems = [g0, g1, g2]
        ssems = [s0, s1, s2]
        c = lax.axis_index("c")
        s = lax.axis_index("s")
        wid = s * NC + c

        pltpu.sync_copy(src_hbm.at[wid], sidx)
        pltpu.sync_copy(dst_hbm.at[wid], didx)
        row0 = s * ROWS_PER_TILE
        pltpu.sync_copy(zeros_hbm, agg_sh.at[pl.ds(row0, ROWS_PER_TILE)])
        plsc.subcore_barrier()

        for b in range(_NBUF):
            pltpu.async_copy(h_hbm.at[sidx.at[b]], bufs[b], gsems[b])

        @pl.loop(0, NCHUNKS_AGG, step=_NBUF)
        def _(jj):
            for b in range(_NBUF):
                pltpu.make_async_copy(h_hbm.at[sidx.at[jj + b]],
                                      bufs[b], gsems[b]).wait()
                pltpu.async_copy(bufs[b], agg_sh.at[didx.at[jj + b]],
                                 ssems[b], add=True)
            for b in range(_NBUF):
                pltpu.make_async_copy(bufs[b], agg_sh.at[didx.at[jj + b]],
                                      ssems[b]).wait()

                @pl.when(jj + _NBUF + b < NCHUNKS_AGG)
                def _(b=b, jj=jj):
                    pltpu.async_copy(h_hbm.at[sidx.at[jj + _NBUF + b]],
                                     bufs[b], gsems[b])

        plsc.subcore_barrier()
        pltpu.sync_copy(agg_sh.at[pl.ds(row0, ROWS_PER_TILE)],
                        out_hbm.at[c, pl.ds(row0, ROWS_PER_TILE)])

    return _sc_aggregate


_sc_aggregate_h = _make_sc_aggregate(HIDDEN)
_sc_aggregate_c = _make_sc_aggregate(CLS_PAD)


_ROWS_BLK = 1000
_GRID = N_NODES // _ROWS_BLK
_ROWS_BLK_P = 640
_GRID_P = N_PAD // _ROWS_BLK_P


def _norm_from_partials(p_ref):
    deg = p_ref[0][:, :1] + p_ref[1][:, :1]
    return lax.rsqrt(jnp.maximum(deg, 1.0))


def _row_mask(i):
    row = i * _ROWS_BLK_P + lax.broadcasted_iota(jnp.int32, (_ROWS_BLK_P, 1), 0)
    return row < N_NODES


def _tc_scale_body(feat_ref, odp_ref, h1_ref):
    i = pl.program_id(0)
    h1 = feat_ref[...] * _norm_from_partials(odp_ref)
    h1_ref[...] = jnp.where(_row_mask(i), h1, 0.0)


def _tc_scale(features, odeg_p):
    return pl.pallas_call(
        _tc_scale_body,
        grid=(_GRID_P,),
        in_specs=[
            pl.BlockSpec((_ROWS_BLK_P, IN_FEATS), lambda i: (i, 0)),
            pl.BlockSpec((NC, _ROWS_BLK_P, 16), lambda i: (0, i, 0)),
        ],
        out_specs=pl.BlockSpec((_ROWS_BLK_P, IN_FEATS), lambda i: (i, 0)),
        out_shape=jax.ShapeDtypeStruct((N_PAD, IN_FEATS), jnp.float32),
    )(features, odeg_p)


def _tc_layer1_body(p1_ref, idp_ref, odp_ref, w1_ref, b1_ref, w2_ref, y_ref):
    i = pl.program_id(0)
    agg = (p1_ref[0] + p1_ref[1]) * _norm_from_partials(idp_ref)
    x1 = jnp.dot(agg, w1_ref[...], preferred_element_type=jnp.float32,
                 precision=lax.Precision.HIGHEST)
    x1 = jnp.maximum(x1 + b1_ref[...], 0.0)
    x1 = x1 * _norm_from_partials(odp_ref)
    y = jnp.dot(x1, w2_ref[...], preferred_element_type=jnp.float32,
                precision=lax.Precision.HIGHEST)
    y_ref[...] = jnp.where(_row_mask(i), y, 0.0)


def _tc_layer1(p1, ideg_p, odeg_p, W1, b1, W2p):
    return pl.pallas_call(
        _tc_layer1_body,
        grid=(_GRID_P,),
        in_specs=[
            pl.BlockSpec((NC, _ROWS_BLK_P, HIDDEN), lambda i: (0, i, 0)),
            pl.BlockSpec((NC, _ROWS_BLK_P, 16), lambda i: (0, i, 0)),
            pl.BlockSpec((NC, _ROWS_BLK_P, 16), lambda i: (0, i, 0)),
            pl.BlockSpec((IN_FEATS, HIDDEN), lambda i: (0, 0)),
            pl.BlockSpec((1, HIDDEN), lambda i: (0, 0)),
            pl.BlockSpec((HIDDEN, CLS_PAD), lambda i: (0, 0)),
        ],
        out_specs=pl.BlockSpec((_ROWS_BLK_P, CLS_PAD), lambda i: (i, 0)),
        out_shape=jax.ShapeDtypeStruct((N_PAD, CLS_PAD), jnp.float32),
    )(p1, ideg_p, odeg_p, W1, b1, W2p)


def _tc_layer2_body(p2_ref, idp_ref, b2_ref, out_ref):
    agg = (p2_ref[0] + p2_ref[1])[:, :NUM_CLASSES]
    out_ref[...] = agg * _norm_from_partials(idp_ref) + b2_ref[...]


def _tc_layer2(p2, ideg_p, b2):
    return pl.pallas_call(
        _tc_layer2_body,
        grid=(_GRID,),
        in_specs=[
            pl.BlockSpec((NC, _ROWS_BLK, CLS_PAD), lambda i: (0, i, 0)),
            pl.BlockSpec((NC, _ROWS_BLK, 16), lambda i: (0, i, 0)),
            pl.BlockSpec((1, NUM_CLASSES), lambda i: (0, 0)),
        ],
        out_specs=pl.BlockSpec((_ROWS_BLK, NUM_CLASSES), lambda i: (i, 0)),
        out_shape=jax.ShapeDtypeStruct((N_NODES, NUM_CLASSES), jnp.float32),
    )(p2, ideg_p, b2)


def kernel(features, edge_index, W1, b1, W2, b2):
    src = edge_index[0].reshape(NW, NCHUNKS, CHUNK)
    dst = edge_index[1].reshape(NW, NCHUNKS, CHUNK)
    pad = NCHUNKS_AGG * CHUNK_AGG - EDGES_PER_TILE
    src_a = jnp.pad(edge_index[0].reshape(NW, EDGES_PER_TILE),
                    ((0, 0), (0, pad)),
                    constant_values=N_NODES).reshape(NW, NCHUNKS_AGG, CHUNK_AGG)
    dst_a = jnp.pad(edge_index[1].reshape(NW, EDGES_PER_TILE),
                    ((0, 0), (0, pad)),
                    constant_values=N_NODES).reshape(NW, NCHUNKS_AGG, CHUNK_AGG)

    ones16 = jnp.ones((CHUNK, 16), jnp.float32)
    zeros16 = jnp.zeros((ROWS_PER_TILE, 16), jnp.float32)
    zeros_h = jnp.zeros((ROWS_PER_TILE, HIDDEN), jnp.float32)
    zeros_c = jnp.zeros((ROWS_PER_TILE, CLS_PAD), jnp.float32)
    W2p = jnp.pad(W2, ((0, 0), (0, CLS_PAD - NUM_CLASSES)))

    odeg_p, ideg_p = _sc_degrees(src, dst, ones16, zeros16)

    h1 = _tc_scale(features, odeg_p)
    p1 = _sc_aggregate_h(h1, src_a, dst_a, zeros_h)
    y = _tc_layer1(p1, ideg_p, odeg_p, W1, b1.reshape(1, HIDDEN), W2p)
    p2 = _sc_aggregate_c(y, src_a, dst_a, zeros_c)
    out = _tc_layer2(p2, ideg_p, b2.reshape(1, NUM_CLASSES))
    return out


# speedup vs baseline: 1.7181x; 1.7181x over previous
import functools

import jax
import jax.numpy as jnp
from jax import lax
from jax.experimental import pallas as pl
from jax.experimental.pallas import tpu as pltpu
from jax.experimental.pallas import tpu_sc as plsc

N_NODES = 10000
N_EDGES = 320000
IN_FEATS = 128
HIDDEN = 128
NUM_CLASSES = 40
CLS_PAD = 48

NC = 2
NS = 16
NW = NC * NS
EDGES_PER_TILE = N_EDGES // NW
CHUNK = 80
NCHUNKS = EDGES_PER_TILE // CHUNK
N_PAD = 10240
CHUNK_AGG = 72
NCHUNKS_AGG = 141
ROWS_PER_TILE = N_PAD // NS

_mesh = plsc.VectorSubcoreMesh(core_axis_name="c", subcore_axis_name="s")
_sc_params = pltpu.CompilerParams(use_tc_tiling_on_sc=False)


@functools.partial(
    pl.kernel,
    out_type=[
        jax.ShapeDtypeStruct((NC, N_PAD, 16), jnp.float32),
        jax.ShapeDtypeStruct((NC, N_PAD, 16), jnp.float32),
    ],
    mesh=_mesh,
    scratch_types=[
        pltpu.VMEM((NCHUNKS, CHUNK), jnp.int32),
        pltpu.VMEM((NCHUNKS, CHUNK), jnp.int32),
        pltpu.VMEM((CHUNK, 16), jnp.float32),
        pltpu.VMEM_SHARED((N_PAD, 16), jnp.float32),
        pltpu.SemaphoreType.DMA,
    ],
    compiler_params=_sc_params,
)
def _sc_degrees(src_hbm, dst_hbm, ones_hbm, zeros_hbm, od_out, id_out,
                sidx, didx, ones_v, deg_sh, sem):
    c = lax.axis_index("c")
    s = lax.axis_index("s")
    wid = s * NC + c

    pltpu.sync_copy(src_hbm.at[wid], sidx)
    pltpu.sync_copy(dst_hbm.at[wid], didx)
    pltpu.sync_copy(ones_hbm, ones_v)
    row0 = s * ROWS_PER_TILE

    for idx_ref, out_ref in ((sidx, od_out), (didx, id_out)):
        pltpu.sync_copy(zeros_hbm, deg_sh.at[pl.ds(row0, ROWS_PER_TILE)])
        plsc.subcore_barrier()

        @pl.loop(0, NCHUNKS)
        def _(j, idx_ref=idx_ref):
            pltpu.async_copy(ones_v, deg_sh.at[idx_ref.at[j]], sem, add=True)

            @pl.when(j >= 4)
            def _():
                pltpu.make_async_copy(ones_v, deg_sh.at[idx_ref.at[j - 4]],
                                      sem).wait()

        @pl.loop(NCHUNKS - 4, NCHUNKS)
        def _(j, idx_ref=idx_ref):
            pltpu.make_async_copy(ones_v, deg_sh.at[idx_ref.at[j]], sem).wait()

        plsc.subcore_barrier()
        pltpu.sync_copy(deg_sh.at[pl.ds(row0, ROWS_PER_TILE)],
                        out_ref.at[c, pl.ds(row0, ROWS_PER_TILE)])
        plsc.subcore_barrier()


_NBUF = 3


def _make_sc_aggregate(width):
    @functools.partial(
        pl.kernel,
        out_type=jax.ShapeDtypeStruct((NC, N_PAD, width), jnp.float32),
        mesh=_mesh,
        scratch_types=[
            pltpu.VMEM((NCHUNKS_AGG, CHUNK_AGG), jnp.int32),
            pltpu.VMEM((NCHUNKS_AGG, CHUNK_AGG), jnp.int32),
            pltpu.VMEM((CHUNK_AGG, width), jnp.float32),
            pltpu.VMEM((CHUNK_AGG, width), jnp.float32),
            pltpu.VMEM((CHUNK_AGG, width), jnp.float32),
            pltpu.SemaphoreType.DMA,
            pltpu.SemaphoreType.DMA,
            pltpu.SemaphoreType.DMA,
            pltpu.SemaphoreType.DMA,
            pltpu.SemaphoreType.DMA,
            pltpu.SemaphoreType.DMA,
            pltpu.VMEM_SHARED((N_PAD, width), jnp.float32),
        ],
        compiler_params=_sc_params,
    )
    def _sc_aggregate(h_hbm, src_hbm, dst_hbm, zeros_hbm, out_hbm,
                      sidx, didx, b0, b1, b2,
                      g0, g1, g2, s0, s1, s2, agg_sh):
        bufs = [b0, b1, b2]
        gsems = [g0, g1, g2]
        ssems = [s0, s1, s2]
        c = lax.axis_index("c")
        s = lax.axis_index("s")
        wid = s * NC + c

        pltpu.sync_copy(src_hbm.at[wid], sidx)
        pltpu.sync_copy(dst_hbm.at[wid], didx)
        row0 = s * ROWS_PER_TILE
        pltpu.sync_copy(zeros_hbm, agg_sh.at[pl.ds(row0, ROWS_PER_TILE)])
        plsc.subcore_barrier()

        for b in range(_NBUF):
            pltpu.async_copy(h_hbm.at[sidx.at[b]], bufs[b], gsems[b])

        @pl.loop(0, NCHUNKS_AGG, step=_NBUF)
        def _(jj):
            for b in range(_NBUF):
                pltpu.make_async_copy(h_hbm.at[sidx.at[jj + b]],
                                      bufs[b], gsems[b]).wait()
                pltpu.async_copy(bufs[b], agg_sh.at[didx.at[jj + b]],
                                 ssems[b], add=True)
            for b in range(_NBUF):
                pltpu.make_async_copy(bufs[b], agg_sh.at[didx.at[jj + b]],
                                      ssems[b]).wait()

                @pl.when(jj + _NBUF + b < NCHUNKS_AGG)
                def _(b=b, jj=jj):
                    pltpu.async_copy(h_hbm.at[sidx.at[jj + _NBUF + b]],
                                     bufs[b], gsems[b])

        plsc.subcore_barrier()
        pltpu.sync_copy(agg_sh.at[pl.ds(row0, ROWS_PER_TILE)],
                        out_hbm.at[c, pl.ds(row0, ROWS_PER_TILE)])

    return _sc_aggregate


_sc_aggregate_h = _make_sc_aggregate(HIDDEN)
_sc_aggregate_c = _make_sc_aggregate(CLS_PAD)


_ROWS_BLK = 1000
_GRID = N_NODES // _ROWS_BLK
_ROWS_BLK_P = 640
_GRID_P = N_PAD // _ROWS_BLK_P


def _norm_from_partials(p_ref):
    deg = p_ref[0][:, :1] + p_ref[1][:, :1]
    return lax.rsqrt(jnp.maximum(deg, 1.0))


def _row_mask(i):
    row = i * _ROWS_BLK_P + lax.broadcasted_iota(jnp.int32, (_ROWS_BLK_P, 1), 0)
    return row < N_NODES


def _tc_scale_body(feat_ref, odp_ref, h1_ref):
    i = pl.program_id(0)
    h1 = feat_ref[...] * _norm_from_partials(odp_ref)
    h1_ref[...] = jnp.where(_row_mask(i), h1, 0.0)


def _tc_scale(features, odeg_p):
    return pl.pallas_call(
        _tc_scale_body,
        grid=(_GRID_P,),
        in_specs=[
            pl.BlockSpec((_ROWS_BLK_P, IN_FEATS), lambda i: (i, 0)),
            pl.BlockSpec((NC, _ROWS_BLK_P, 16), lambda i: (0, i, 0)),
        ],
        out_specs=pl.BlockSpec((_ROWS_BLK_P, IN_FEATS), lambda i: (i, 0)),
        out_shape=jax.ShapeDtypeStruct((N_PAD, IN_FEATS), jnp.float32),
    )(features, odeg_p)


def _tc_layer1_body(p1_ref, idp_ref, odp_ref, w1_ref, b1_ref, w2_ref, y_ref):
    i = pl.program_id(0)
    agg = (p1_ref[0] + p1_ref[1]) * _norm_from_partials(idp_ref)
    x1 = jnp.dot(agg, w1_ref[...], preferred_element_type=jnp.float32,
                 precision=lax.Precision.HIGHEST)
    x1 = jnp.maximum(x1 + b1_ref[...], 0.0)
    x1 = x1 * _norm_from_partials(odp_ref)
    y = jnp.dot(x1, w2_ref[...], preferred_element_type=jnp.float32,
                precision=lax.Precision.HIGHEST)
    y_ref[...] = jnp.where(_row_mask(i), y, 0.0)


def _tc_layer1(p1, ideg_p, odeg_p, W1, b1, W2p):
    return pl.pallas_call(
        _tc_layer1_body,
        grid=(_GRID_P,),
        in_specs=[
            pl.BlockSpec((NC, _ROWS_BLK_P, HIDDEN), lambda i: (0, i, 0)),
            pl.BlockSpec((NC, _ROWS_BLK_P, 16), lambda i: (0, i, 0)),
            pl.BlockSpec((NC, _ROWS_BLK_P, 16), lambda i: (0, i, 0)),
            pl.BlockSpec((IN_FEATS, HIDDEN), lambda i: (0, 0)),
            pl.BlockSpec((1, HIDDEN), lambda i: (0, 0)),
            pl.BlockSpec((HIDDEN, CLS_PAD), lambda i: (0, 0)),
        ],
        out_specs=pl.BlockSpec((_ROWS_BLK_P, CLS_PAD), lambda i: (i, 0)),
        out_shape=jax.ShapeDtypeStruct((N_PAD, CLS_PAD), jnp.float32),
    )(p1, ideg_p, odeg_p, W1, b1, W2p)


def _tc_layer2_body(p2_ref, idp_ref, b2_ref, out_ref):
    agg = (p2_ref[0] + p2_ref[1])[:, :NUM_CLASSES]
    out_ref[...] = agg * _norm_from_partials(idp_ref) + b2_ref[...]


def _tc_layer2(p2, ideg_p, b2):
    return pl.pallas_call(
        _tc_layer2_body,
        grid=(_GRID,),
        in_specs=[
            pl.BlockSpec((NC, _ROWS_BLK, CLS_PAD), lambda i: (0, i, 0)),
            pl.BlockSpec((NC, _ROWS_BLK, 16), lambda i: (0, i, 0)),
            pl.BlockSpec((1, NUM_CLASSES), lambda i: (0, 0)),
        ],
        out_specs=pl.BlockSpec((_ROWS_BLK, NUM_CLASSES), lambda i: (i, 0)),
        out_shape=jax.ShapeDtypeStruct((N_NODES, NUM_CLASSES), jnp.float32),
    )(p2, ideg_p, b2)


def kernel(features, edge_index, W1, b1, W2, b2):
    src = edge_index[0].reshape(NW, NCHUNKS, CHUNK)
    dst = edge_index[1].reshape(NW, NCHUNKS, CHUNK)
    pad = NCHUNKS_AGG * CHUNK_AGG - EDGES_PER_TILE
    pad_rows = N_NODES + (jnp.arange(NW * pad, dtype=jnp.int32) % (N_PAD - N_NODES))
    pad_rows = pad_rows.reshape(NW, pad)
    src_a = jnp.concatenate(
        [edge_index[0].reshape(NW, EDGES_PER_TILE), pad_rows], axis=1
    ).reshape(NW, NCHUNKS_AGG, CHUNK_AGG)
    dst_a = jnp.concatenate(
        [edge_index[1].reshape(NW, EDGES_PER_TILE), pad_rows], axis=1
    ).reshape(NW, NCHUNKS_AGG, CHUNK_AGG)

    ones16 = jnp.ones((CHUNK, 16), jnp.float32)
    zeros16 = jnp.zeros((ROWS_PER_TILE, 16), jnp.float32)
    zeros_h = jnp.zeros((ROWS_PER_TILE, HIDDEN), jnp.float32)
    zeros_c = jnp.zeros((ROWS_PER_TILE, CLS_PAD), jnp.float32)
    W2p = jnp.pad(W2, ((0, 0), (0, CLS_PAD - NUM_CLASSES)))

    odeg_p, ideg_p = _sc_degrees(src, dst, ones16, zeros16)

    h1 = _tc_scale(features, odeg_p)
    p1 = _sc_aggregate_h(h1, src_a, dst_a, zeros_h)
    y = _tc_layer1(p1, ideg_p, odeg_p, W1, b1.reshape(1, HIDDEN), W2p)
    p2 = _sc_aggregate_c(y, src_a, dst_a, zeros_c)
    out = _tc_layer2(p2, ideg_p, b2.reshape(1, NUM_CLASSES))
    return out


# speedup vs baseline: 1.7242x; 1.0036x over previous
import functools

import jax
import jax.numpy as jnp
from jax import lax
from jax.experimental import pallas as pl
from jax.experimental.pallas import tpu as pltpu
from jax.experimental.pallas import tpu_sc as plsc

N_NODES = 10000
N_EDGES = 320000
IN_FEATS = 128
HIDDEN = 128
NUM_CLASSES = 40
CLS_PAD = 48

NC = 2
NS = 16
NW = NC * NS
EDGES_PER_TILE = N_EDGES // NW
CHUNK = 80
NCHUNKS = EDGES_PER_TILE // CHUNK
N_PAD = 10240
CHUNK_AGG = 72
NCHUNKS_AGG = 141
ROWS_PER_TILE = N_PAD // NS

_mesh = plsc.VectorSubcoreMesh(core_axis_name="c", subcore_axis_name="s")
_sc_params = pltpu.CompilerParams(use_tc_tiling_on_sc=False)


@functools.partial(
    pl.kernel,
    out_type=[
        jax.ShapeDtypeStruct((NC, N_PAD, 16), jnp.float32),
        jax.ShapeDtypeStruct((NC, N_PAD, 16), jnp.float32),
    ],
    mesh=_mesh,
    scratch_types=[
        pltpu.VMEM((NCHUNKS, CHUNK), jnp.int32),
        pltpu.VMEM((NCHUNKS, CHUNK), jnp.int32),
        pltpu.VMEM((CHUNK, 16), jnp.float32),
        pltpu.VMEM_SHARED((N_PAD, 16), jnp.float32),
        pltpu.VMEM_SHARED((N_PAD, 16), jnp.float32),
        pltpu.SemaphoreType.DMA,
        pltpu.SemaphoreType.DMA,
    ],
    compiler_params=_sc_params,
)
def _sc_degrees(src_hbm, dst_hbm, ones_hbm, zeros_hbm, od_out, id_out,
                sidx, didx, ones_v, od_sh, id_sh, sem_o, sem_i):
    c = lax.axis_index("c")
    s = lax.axis_index("s")
    wid = s * NC + c

    pltpu.sync_copy(src_hbm.at[wid], sidx)
    pltpu.sync_copy(dst_hbm.at[wid], didx)
    pltpu.sync_copy(ones_hbm, ones_v)
    row0 = s * ROWS_PER_TILE
    pltpu.sync_copy(zeros_hbm, od_sh.at[pl.ds(row0, ROWS_PER_TILE)])
    pltpu.sync_copy(zeros_hbm, id_sh.at[pl.ds(row0, ROWS_PER_TILE)])
    plsc.subcore_barrier()

    @pl.loop(0, NCHUNKS)
    def _(j):
        pltpu.async_copy(ones_v, od_sh.at[sidx.at[j]], sem_o, add=True)
        pltpu.async_copy(ones_v, id_sh.at[didx.at[j]], sem_i, add=True)

        @pl.when(j >= 4)
        def _():
            pltpu.make_async_copy(ones_v, od_sh.at[sidx.at[j - 4]], sem_o).wait()
            pltpu.make_async_copy(ones_v, id_sh.at[didx.at[j - 4]], sem_i).wait()

    @pl.loop(NCHUNKS - 4, NCHUNKS)
    def _(j):
        pltpu.make_async_copy(ones_v, od_sh.at[sidx.at[j]], sem_o).wait()
        pltpu.make_async_copy(ones_v, id_sh.at[didx.at[j]], sem_i).wait()

    plsc.subcore_barrier()
    pltpu.sync_copy(od_sh.at[pl.ds(row0, ROWS_PER_TILE)],
                    od_out.at[c, pl.ds(row0, ROWS_PER_TILE)])
    pltpu.sync_copy(id_sh.at[pl.ds(row0, ROWS_PER_TILE)],
                    id_out.at[c, pl.ds(row0, ROWS_PER_TILE)])


_NBUF = 3


def _make_sc_aggregate(width):
    @functools.partial(
        pl.kernel,
        out_type=jax.ShapeDtypeStruct((NC, N_PAD, width), jnp.float32),
        mesh=_mesh,
        scratch_types=[
            pltpu.VMEM((NCHUNKS_AGG, CHUNK_AGG), jnp.int32),
            pltpu.VMEM((NCHUNKS_AGG, CHUNK_AGG), jnp.int32),
            pltpu.VMEM((CHUNK_AGG, width), jnp.float32),
            pltpu.VMEM((CHUNK_AGG, width), jnp.float32),
            pltpu.VMEM((CHUNK_AGG, width), jnp.float32),
            pltpu.SemaphoreType.DMA,
            pltpu.SemaphoreType.DMA,
            pltpu.SemaphoreType.DMA,
            pltpu.SemaphoreType.DMA,
            pltpu.SemaphoreType.DMA,
            pltpu.SemaphoreType.DMA,
            pltpu.VMEM_SHARED((N_PAD, width), jnp.float32),
        ],
        compiler_params=_sc_params,
    )
    def _sc_aggregate(h_hbm, src_hbm, dst_hbm, zeros_hbm, out_hbm,
                      sidx, didx, b0, b1, b2,
                      g0, g1, g2, s0, s1, s2, agg_sh):
        bufs = [b0, b1, b2]
        gsems = [g0, g1, g2]
        ssems = [s0, s1, s2]
        c = lax.axis_index("c")
        s = lax.axis_index("s")
        wid = s * NC + c

        pltpu.sync_copy(src_hbm.at[wid], sidx)
        pltpu.sync_copy(dst_hbm.at[wid], didx)
        row0 = s * ROWS_PER_TILE
        pltpu.sync_copy(zeros_hbm, agg_sh.at[pl.ds(row0, ROWS_PER_TILE)])
        plsc.subcore_barrier()

        for b in range(_NBUF):
            pltpu.async_copy(h_hbm.at[sidx.at[b]], bufs[b], gsems[b])

        @pl.loop(0, NCHUNKS_AGG, step=_NBUF)
        def _(jj):
            for b in range(_NBUF):
                pltpu.make_async_copy(h_hbm.at[sidx.at[jj + b]],
                                      bufs[b], gsems[b]).wait()
                pltpu.async_copy(bufs[b], agg_sh.at[didx.at[jj + b]],
                                 ssems[b], add=True)
            for b in range(_NBUF):
                pltpu.make_async_copy(bufs[b], agg_sh.at[didx.at[jj + b]],
                                      ssems[b]).wait()

                @pl.when(jj + _NBUF + b < NCHUNKS_AGG)
                def _(b=b, jj=jj):
                    pltpu.async_copy(h_hbm.at[sidx.at[jj + _NBUF + b]],
                                     bufs[b], gsems[b])

        plsc.subcore_barrier()
        pltpu.sync_copy(agg_sh.at[pl.ds(row0, ROWS_PER_TILE)],
                        out_hbm.at[c, pl.ds(row0, ROWS_PER_TILE)])

    return _sc_aggregate


_sc_aggregate_h = _make_sc_aggregate(HIDDEN)
_sc_aggregate_c = _make_sc_aggregate(CLS_PAD)


_ROWS_BLK = 1000
_GRID = N_NODES // _ROWS_BLK
_ROWS_BLK_P = 640
_GRID_P = N_PAD // _ROWS_BLK_P


def _norm_from_partials(p_ref):
    deg = p_ref[0][:, :1] + p_ref[1][:, :1]
    return lax.rsqrt(jnp.maximum(deg, 1.0))


def _row_mask(i):
    row = i * _ROWS_BLK_P + lax.broadcasted_iota(jnp.int32, (_ROWS_BLK_P, 1), 0)
    return row < N_NODES


def _tc_scale_body(feat_ref, odp_ref, h1_ref):
    i = pl.program_id(0)
    h1 = feat_ref[...] * _norm_from_partials(odp_ref)
    h1_ref[...] = jnp.where(_row_mask(i), h1, 0.0)


def _tc_scale(features, odeg_p):
    return pl.pallas_call(
        _tc_scale_body,
        grid=(_GRID_P,),
        in_specs=[
            pl.BlockSpec((_ROWS_BLK_P, IN_FEATS), lambda i: (i, 0)),
            pl.BlockSpec((NC, _ROWS_BLK_P, 16), lambda i: (0, i, 0)),
        ],
        out_specs=pl.BlockSpec((_ROWS_BLK_P, IN_FEATS), lambda i: (i, 0)),
        out_shape=jax.ShapeDtypeStruct((N_PAD, IN_FEATS), jnp.float32),
    )(features, odeg_p)


def _tc_layer1_body(p1_ref, idp_ref, odp_ref, w1_ref, b1_ref, w2_ref, y_ref):
    i = pl.program_id(0)
    agg = (p1_ref[0] + p1_ref[1]) * _norm_from_partials(idp_ref)
    x1 = jnp.dot(agg, w1_ref[...], preferred_element_type=jnp.float32,
                 precision=lax.Precision.HIGHEST)
    x1 = jnp.maximum(x1 + b1_ref[...], 0.0)
    x1 = x1 * _norm_from_partials(odp_ref)
    y = jnp.dot(x1, w2_ref[...], preferred_element_type=jnp.float32,
                precision=lax.Precision.HIGHEST)
    y_ref[...] = jnp.where(_row_mask(i), y, 0.0)


def _tc_layer1(p1, ideg_p, odeg_p, W1, b1, W2p):
    return pl.pallas_call(
        _tc_layer1_body,
        grid=(_GRID_P,),
        in_specs=[
            pl.BlockSpec((NC, _ROWS_BLK_P, HIDDEN), lambda i: (0, i, 0)),
            pl.BlockSpec((NC, _ROWS_BLK_P, 16), lambda i: (0, i, 0)),
            pl.BlockSpec((NC, _ROWS_BLK_P, 16), lambda i: (0, i, 0)),
            pl.BlockSpec((IN_FEATS, HIDDEN), lambda i: (0, 0)),
            pl.BlockSpec((1, HIDDEN), lambda i: (0, 0)),
            pl.BlockSpec((HIDDEN, CLS_PAD), lambda i: (0, 0)),
        ],
        out_specs=pl.BlockSpec((_ROWS_BLK_P, CLS_PAD), lambda i: (i, 0)),
        out_shape=jax.ShapeDtypeStruct((N_PAD, CLS_PAD), jnp.float32),
    )(p1, ideg_p, odeg_p, W1, b1, W2p)


def _tc_layer2_body(p2_ref, idp_ref, b2_ref, out_ref):
    agg = (p2_ref[0] + p2_ref[1])[:, :NUM_CLASSES]
    out_ref[...] = agg * _norm_from_partials(idp_ref) + b2_ref[...]


def _tc_layer2(p2, ideg_p, b2):
    return pl.pallas_call(
        _tc_layer2_body,
        grid=(_GRID,),
        in_specs=[
            pl.BlockSpec((NC, _ROWS_BLK, CLS_PAD), lambda i: (0, i, 0)),
            pl.BlockSpec((NC, _ROWS_BLK, 16), lambda i: (0, i, 0)),
            pl.BlockSpec((1, NUM_CLASSES), lambda i: (0, 0)),
        ],
        out_specs=pl.BlockSpec((_ROWS_BLK, NUM_CLASSES), lambda i: (i, 0)),
        out_shape=jax.ShapeDtypeStruct((N_NODES, NUM_CLASSES), jnp.float32),
    )(p2, ideg_p, b2)


def kernel(features, edge_index, W1, b1, W2, b2):
    src = edge_index[0].reshape(NW, NCHUNKS, CHUNK)
    dst = edge_index[1].reshape(NW, NCHUNKS, CHUNK)
    pad = NCHUNKS_AGG * CHUNK_AGG - EDGES_PER_TILE
    pad_rows = N_NODES + (jnp.arange(NW * pad, dtype=jnp.int32) % (N_PAD - N_NODES))
    pad_rows = pad_rows.reshape(NW, pad)
    src_a = jnp.concatenate(
        [edge_index[0].reshape(NW, EDGES_PER_TILE), pad_rows], axis=1
    ).reshape(NW, NCHUNKS_AGG, CHUNK_AGG)
    dst_a = jnp.concatenate(
        [edge_index[1].reshape(NW, EDGES_PER_TILE), pad_rows], axis=1
    ).reshape(NW, NCHUNKS_AGG, CHUNK_AGG)

    ones16 = jnp.ones((CHUNK, 16), jnp.float32)
    zeros16 = jnp.zeros((ROWS_PER_TILE, 16), jnp.float32)
    zeros_h = jnp.zeros((ROWS_PER_TILE, HIDDEN), jnp.float32)
    zeros_c = jnp.zeros((ROWS_PER_TILE, CLS_PAD), jnp.float32)
    W2p = jnp.pad(W2, ((0, 0), (0, CLS_PAD - NUM_CLASSES)))

    odeg_p, ideg_p = _sc_degrees(src, dst, ones16, zeros16)

    h1 = _tc_scale(features, odeg_p)
    p1 = _sc_aggregate_h(h1, src_a, dst_a, zeros_h)
    y = _tc_layer1(p1, ideg_p, odeg_p, W1, b1.reshape(1, HIDDEN), W2p)
    p2 = _sc_aggregate_c(y, src_a, dst_a, zeros_c)
    out = _tc_layer2(p2, ideg_p, b2.reshape(1, NUM_CLASSES))
    return out
